# baseline (speedup 1.0000x reference)
import functools

import numpy as np
import jax
import jax.numpy as jnp
from jax import lax
from jax.experimental import pallas as pl
from jax.experimental.pallas import tpu as pltpu
from jax.experimental.pallas import tpu_sc as plsc

_MAX_LEN = 2048
_MODEL_DIM = 1024
_BATCH = 4

_NC = 2
_NS = 16
_NW = _NC * _NS
_POS_PER_TILE = _MAX_LEN // _NW
_PW = 32
_CHUNKS = _POS_PER_TILE // _PW
_LANES = 16


def _pe_np():
    pos = np.arange(_MAX_LEN)[:, None]
    pe = pos / np.power(10000, 2.0 * np.arange(_MODEL_DIM)[None, :] / _MODEL_DIM)
    pe[:, 0::2] = np.sin(pe[:, 0::2])
    pe[:, 1::2] = np.cos(pe[:, 1::2])
    return pe.astype(np.float32)


_PE = _pe_np()


_mesh = plsc.VectorSubcoreMesh(core_axis_name="c", subcore_axis_name="s")

_out_struct = jax.ShapeDtypeStruct((_BATCH * _MAX_LEN, _MODEL_DIM), jnp.float32)


@functools.partial(
    pl.kernel,
    mesh=_mesh,
    out_type=(_out_struct, _out_struct),
    scratch_types=[
        pltpu.VMEM((_PW,), jnp.int32),
        pltpu.VMEM((_PW, _MODEL_DIM), jnp.float32),
        pltpu.VMEM((_PW, _MODEL_DIM), jnp.float32),
        pltpu.SemaphoreType.DMA,
    ],
)
def _embed_kernel(table_hbm, idx_hbm, pe_hbm, ox_hbm, oy_hbm,
                  idx_v, pe_v, rows_v, sem):
    wid = lax.axis_index("s") * _NC + lax.axis_index("c")
    t_base = wid * _POS_PER_TILE
    for pc in range(_CHUNKS):
        t0 = t_base + pc * _PW
        pltpu.sync_copy(pe_hbm.at[pl.ds(t0, _PW)], pe_v)
        for tensor in range(2):
            out_hbm = ox_hbm if tensor == 0 else oy_hbm
            for b in range(_BATCH):
                off = tensor * (_BATCH * _MAX_LEN) + b * _MAX_LEN
                pltpu.sync_copy(idx_hbm.at[pl.ds(off + t0, _PW)], idx_v)
                pltpu.async_copy(table_hbm.at[idx_v], rows_v, sem).wait()

                @pl.loop(0, _PW)
                def _(r):
                    @pl.loop(0, _MODEL_DIM, step=_LANES)
                    def _(c):
                        slc = (pl.ds(r, 1), pl.ds(c, _LANES))
                        rows_v.at[*slc][...] = (
                            rows_v.at[*slc][...] + pe_v.at[*slc][...]
                        )

                pltpu.sync_copy(rows_v, out_hbm.at[pl.ds(b * _MAX_LEN + t0, _PW)])


def kernel(x, y, table):
    idx = jnp.concatenate(
        [x.reshape(-1), y[:, :-1].reshape(-1)]).astype(jnp.int32)
    pe = jnp.asarray(_PE)
    ox, oy = _embed_kernel(table, idx, pe)
    return (ox.reshape(_BATCH, _MAX_LEN, _MODEL_DIM),
            oy.reshape(_BATCH, _MAX_LEN, _MODEL_DIM))
